# initial kernel scaffold (unmeasured)
import jax
import jax.numpy as jnp
from jax import lax
from jax.experimental import pallas as pl
from jax.experimental.pallas import tpu as pltpu

N_DEV = 4
DH = 64


def kernel(x, Wq, Wk, Wv, Wo):
    B, S, D = x.shape
    H = Wq.shape[1] // DH

    def body(x_ref, wq_ref, wk_ref, wv_ref, wo_ref, out_ref,
             xg_ref, part_ref, sbuf_ref, rbuf_ref,
             s1_send, s1_recv, s2_send, s2_recv):
        my = lax.axis_index("i")
        left = lax.rem(my + N_DEV - 1, N_DEV)
        right = lax.rem(my + 1, N_DEV)

        barrier = pltpu.get_barrier_semaphore()
        for nbr in (left, right):
            pl.semaphore_signal(barrier, inc=1, device_id=(nbr,),
                                device_id_type=pl.DeviceIdType.MESH)
        pl.semaphore_wait(barrier, 2)

        xg_ref[pl.ds(my, 1)] = x_ref[...][None]
        for h in range(N_DEV - 1):
            send_slot = lax.rem(my - h + N_DEV, N_DEV)
            recv_slot = lax.rem(my - 1 - h + 2 * N_DEV, N_DEV)
            send = pltpu.make_async_remote_copy(
                src_ref=xg_ref.at[send_slot],
                dst_ref=xg_ref.at[send_slot],
                send_sem=s1_send.at[h],
                recv_sem=s1_recv.at[h],
                device_id=(right,),
                device_id_type=pl.DeviceIdType.MESH,
            )
            send.start()
            send.wait_send()
            recv = pltpu.make_async_remote_copy(
                src_ref=xg_ref.at[recv_slot],
                dst_ref=xg_ref.at[recv_slot],
                send_sem=s1_send.at[h],
                recv_sem=s1_recv.at[h],
                device_id=(right,),
                device_id_type=pl.DeviceIdType.MESH,
            )
            recv.wait_recv()

        posf = lax.broadcasted_iota(jnp.float32, (S, DH), 0)
        ji = lax.broadcasted_iota(jnp.int32, (S, DH), 1)
        k2f = ((ji // 2) * 2).astype(jnp.float32)
        inv = jnp.exp(-(k2f / DH) * jnp.log(10000.0))
        ang = posf * inv
        cos = jnp.cos(ang)
        sin = jnp.sin(ang)
        even = (ji % 2) == 0

        def rope(t):
            tl = jnp.concatenate([t[..., 1:], t[..., :1]], axis=-1)
            tr = jnp.concatenate([t[..., -1:], t[..., :-1]], axis=-1)
            t_r = jnp.where(even[None], -tl, tr)
            return t * cos[None] + t_r * sin[None]

        wq = wq_ref[...]
        wk = wk_ref[...]
        wv = wv_ref[...]
        wo = wo_ref[...]
        proj_dn = (((2,), (0,)), ((), ()))
        qk_dn = (((2,), (2,)), ((0,), (0,)))
        av_dn = (((2,), (1,)), ((0,), (0,)))

        for d in range(N_DEV):
            xb = xg_ref[d]
            q = lax.dot_general(xb, wq, proj_dn,
                                preferred_element_type=jnp.float32)
            k = lax.dot_general(xb, wk, proj_dn,
                                preferred_element_type=jnp.float32)
            v = lax.dot_general(xb, wv, proj_dn,
                                preferred_element_type=jnp.float32)
            acc = jnp.zeros((B, S, D), jnp.float32)
            for h in range(H):
                sl = slice(h * DH, (h + 1) * DH)
                qh = rope(q[:, :, sl])
                kh = rope(k[:, :, sl])
                vh = v[:, :, sl]
                s_ = lax.dot_general(qh, kh, qk_dn,
                                     preferred_element_type=jnp.float32)
                s_ = s_ * 0.125
                m = jnp.max(s_, axis=-1, keepdims=True)
                w = jnp.exp(s_ - m)
                w = w / jnp.sum(w, axis=-1, keepdims=True)
                ctx = lax.dot_general(w, vh, av_dn,
                                      preferred_element_type=jnp.float32)
                acc = acc + lax.dot_general(ctx, wo[sl, :], proj_dn,
                                            preferred_element_type=jnp.float32)
            part_ref[d] = acc

        for s in range(N_DEV - 1):
            d_send = lax.rem(my - 1 - s + 2 * N_DEV, N_DEV)
            val = part_ref[pl.ds(d_send, 1)][0]
            if s > 0:
                val = val + rbuf_ref[s - 1]
            sbuf_ref[s] = val
            send = pltpu.make_async_remote_copy(
                src_ref=sbuf_ref.at[s],
                dst_ref=rbuf_ref.at[s],
                send_sem=s2_send.at[s],
                recv_sem=s2_recv.at[s],
                device_id=(right,),
                device_id_type=pl.DeviceIdType.MESH,
            )
            send.start()
            send.wait_send()
            recv = pltpu.make_async_remote_copy(
                src_ref=sbuf_ref.at[s],
                dst_ref=rbuf_ref.at[s],
                send_sem=s2_send.at[s],
                recv_sem=s2_recv.at[s],
                device_id=(right,),
                device_id_type=pl.DeviceIdType.MESH,
            )
            recv.wait_recv()

        out_ref[...] = rbuf_ref[N_DEV - 2] + part_ref[pl.ds(my, 1)][0]

    return pl.pallas_call(
        body,
        out_shape=jax.ShapeDtypeStruct((B, S, D), jnp.float32),
        in_specs=[pl.BlockSpec(memory_space=pltpu.VMEM)] * 5,
        out_specs=pl.BlockSpec(memory_space=pltpu.VMEM),
        scratch_shapes=[
            pltpu.VMEM((N_DEV, B, S, D), jnp.float32),
            pltpu.VMEM((N_DEV, B, S, D), jnp.float32),
            pltpu.VMEM((N_DEV - 1, B, S, D), jnp.float32),
            pltpu.VMEM((N_DEV - 1, B, S, D), jnp.float32),
            pltpu.SemaphoreType.DMA((N_DEV - 1,)),
            pltpu.SemaphoreType.DMA((N_DEV - 1,)),
            pltpu.SemaphoreType.DMA((N_DEV - 1,)),
            pltpu.SemaphoreType.DMA((N_DEV - 1,)),
        ],
        compiler_params=pltpu.CompilerParams(collective_id=0),
    )(x, Wq, Wk, Wv, Wo)


# baseline (device time: 64412 ns/iter reference)
import jax
import jax.numpy as jnp
from jax import lax
from jax.experimental import pallas as pl
from jax.experimental.pallas import tpu as pltpu

N_DEV = 4
DH = 64


def kernel(x, Wq, Wk, Wv, Wo):
    B, S, D = x.shape
    H = Wq.shape[1] // DH

    def body(x_ref, wq_ref, wk_ref, wv_ref, wo_ref, out_ref,
             xg_ref, part_ref, sbuf_ref, rbuf_ref,
             s1_send, s1_recv, s2_send, s2_recv):
        my = lax.axis_index("i")
        left = lax.rem(my + N_DEV - 1, N_DEV)
        right = lax.rem(my + 1, N_DEV)

        barrier = pltpu.get_barrier_semaphore()
        for nbr in (left, right):
            pl.semaphore_signal(barrier, inc=1, device_id=(nbr,),
                                device_id_type=pl.DeviceIdType.MESH)
        pl.semaphore_wait(barrier, 2)

        xg_ref[pl.ds(my, 1)] = x_ref[...][None]
        for h in range(N_DEV - 1):
            send_slot = lax.rem(my - h + N_DEV, N_DEV)
            recv_slot = lax.rem(my - 1 - h + 2 * N_DEV, N_DEV)
            send = pltpu.make_async_remote_copy(
                src_ref=xg_ref.at[send_slot],
                dst_ref=xg_ref.at[send_slot],
                send_sem=s1_send.at[h],
                recv_sem=s1_recv.at[h],
                device_id=(right,),
                device_id_type=pl.DeviceIdType.MESH,
            )
            send.start()
            send.wait_send()
            recv = pltpu.make_async_remote_copy(
                src_ref=xg_ref.at[recv_slot],
                dst_ref=xg_ref.at[recv_slot],
                send_sem=s1_send.at[h],
                recv_sem=s1_recv.at[h],
                device_id=(right,),
                device_id_type=pl.DeviceIdType.MESH,
            )
            recv.wait_recv()

        posf = lax.broadcasted_iota(jnp.int32, (S, DH), 0).astype(jnp.float32)
        ji = lax.broadcasted_iota(jnp.int32, (S, DH), 1)
        k2f = ((ji // 2) * 2).astype(jnp.float32)
        inv = jnp.exp(-(k2f / DH) * jnp.log(10000.0))
        ang = posf * inv
        cos = jnp.cos(ang)
        sin = jnp.sin(ang)
        even = (ji % 2) == 0

        def rope(t):
            tl = jnp.concatenate([t[..., 1:], t[..., :1]], axis=-1)
            tr = jnp.concatenate([t[..., -1:], t[..., :-1]], axis=-1)
            t_r = jnp.where(even[None], -tl, tr)
            return t * cos[None] + t_r * sin[None]

        wq = wq_ref[...]
        wk = wk_ref[...]
        wv = wv_ref[...]
        wo = wo_ref[...]
        proj_dn = (((2,), (0,)), ((), ()))
        qk_dn = (((2,), (2,)), ((0,), (0,)))
        av_dn = (((2,), (1,)), ((0,), (0,)))

        for d in range(N_DEV):
            xb = xg_ref[d]
            q = lax.dot_general(xb, wq, proj_dn,
                                preferred_element_type=jnp.float32)
            k = lax.dot_general(xb, wk, proj_dn,
                                preferred_element_type=jnp.float32)
            v = lax.dot_general(xb, wv, proj_dn,
                                preferred_element_type=jnp.float32)
            acc = jnp.zeros((B, S, D), jnp.float32)
            for h in range(H):
                sl = slice(h * DH, (h + 1) * DH)
                qh = rope(q[:, :, sl])
                kh = rope(k[:, :, sl])
                vh = v[:, :, sl]
                s_ = lax.dot_general(qh, kh, qk_dn,
                                     preferred_element_type=jnp.float32)
                s_ = s_ * 0.125
                m = jnp.max(s_, axis=-1, keepdims=True)
                w = jnp.exp(s_ - m)
                w = w / jnp.sum(w, axis=-1, keepdims=True)
                ctx = lax.dot_general(w, vh, av_dn,
                                      preferred_element_type=jnp.float32)
                acc = acc + lax.dot_general(ctx, wo[sl, :], proj_dn,
                                            preferred_element_type=jnp.float32)
            part_ref[d] = acc

        for s in range(N_DEV - 1):
            d_send = lax.rem(my - 1 - s + 2 * N_DEV, N_DEV)
            val = part_ref[pl.ds(d_send, 1)][0]
            if s > 0:
                val = val + rbuf_ref[s - 1]
            sbuf_ref[s] = val
            send = pltpu.make_async_remote_copy(
                src_ref=sbuf_ref.at[s],
                dst_ref=rbuf_ref.at[s],
                send_sem=s2_send.at[s],
                recv_sem=s2_recv.at[s],
                device_id=(right,),
                device_id_type=pl.DeviceIdType.MESH,
            )
            send.start()
            send.wait_send()
            recv = pltpu.make_async_remote_copy(
                src_ref=sbuf_ref.at[s],
                dst_ref=rbuf_ref.at[s],
                send_sem=s2_send.at[s],
                recv_sem=s2_recv.at[s],
                device_id=(right,),
                device_id_type=pl.DeviceIdType.MESH,
            )
            recv.wait_recv()

        out_ref[...] = rbuf_ref[N_DEV - 2] + part_ref[pl.ds(my, 1)][0]

    return pl.pallas_call(
        body,
        out_shape=jax.ShapeDtypeStruct((B, S, D), jnp.float32),
        in_specs=[pl.BlockSpec(memory_space=pltpu.VMEM)] * 5,
        out_specs=pl.BlockSpec(memory_space=pltpu.VMEM),
        scratch_shapes=[
            pltpu.VMEM((N_DEV, B, S, D), jnp.float32),
            pltpu.VMEM((N_DEV, B, S, D), jnp.float32),
            pltpu.VMEM((N_DEV - 1, B, S, D), jnp.float32),
            pltpu.VMEM((N_DEV - 1, B, S, D), jnp.float32),
            pltpu.SemaphoreType.DMA((N_DEV - 1,)),
            pltpu.SemaphoreType.DMA((N_DEV - 1,)),
            pltpu.SemaphoreType.DMA((N_DEV - 1,)),
            pltpu.SemaphoreType.DMA((N_DEV - 1,)),
        ],
        compiler_params=pltpu.CompilerParams(collective_id=0),
    )(x, Wq, Wk, Wv, Wo)


# device time: 48356 ns/iter; 1.3320x vs baseline; 1.3320x over previous
import jax
import jax.numpy as jnp
from jax import lax
from jax.experimental import pallas as pl
from jax.experimental.pallas import tpu as pltpu

N_DEV = 4
DH = 64


def kernel(x, Wq, Wk, Wv, Wo):
    B, S, D = x.shape
    H = Wq.shape[1] // DH
    HD = H * DH

    def body(x_ref, wq_ref, wk_ref, wv_ref, wo_ref, out_ref,
             xg_ref, pown_ref, sbuf_ref, rbuf_ref,
             ag_send, ag_recv, rs_send, rs_recv):
        my = lax.axis_index("i")
        left = lax.rem(my + N_DEV - 1, N_DEV)
        right = lax.rem(my + 1, N_DEV)

        barrier = pltpu.get_barrier_semaphore()
        for nbr in (left, right):
            pl.semaphore_signal(barrier, inc=1, device_id=(nbr,),
                                device_id_type=pl.DeviceIdType.MESH)
        pl.semaphore_wait(barrier, 2)

        def ag_copy(h):
            src = x_ref if h == 0 else xg_ref.at[h - 1]
            return pltpu.make_async_remote_copy(
                src_ref=src,
                dst_ref=xg_ref.at[h],
                send_sem=ag_send.at[h],
                recv_sem=ag_recv.at[h],
                device_id=(right,),
                device_id_type=pl.DeviceIdType.MESH,
            )

        def rs_copy(s):
            return pltpu.make_async_remote_copy(
                src_ref=sbuf_ref.at[s],
                dst_ref=rbuf_ref.at[s],
                send_sem=rs_send.at[s],
                recv_sem=rs_recv.at[s],
                device_id=(right,),
                device_id_type=pl.DeviceIdType.MESH,
            )

        posf = lax.broadcasted_iota(jnp.int32, (S, HD), 0).astype(jnp.float32)
        ji = lax.broadcasted_iota(jnp.int32, (S, HD), 1)
        jh = ji % DH
        k2f = ((jh // 2) * 2).astype(jnp.float32)
        inv = jnp.exp(-(k2f / DH) * jnp.log(10000.0))
        ang = posf * inv
        cos = jnp.cos(ang)
        sin = jnp.sin(ang)
        even = (ji % 2) == 0

        def rope(t):
            tl = jnp.concatenate([t[..., 1:], t[..., :1]], axis=-1)
            tr = jnp.concatenate([t[..., -1:], t[..., :-1]], axis=-1)
            t_r = jnp.where(even[None], -tl, tr)
            return t * cos[None] + t_r * sin[None]

        wqkv = jnp.concatenate([wq_ref[...], wk_ref[...], wv_ref[...]],
                               axis=1)
        wo = wo_ref[...]
        proj_dn = (((2,), (0,)), ((), ()))
        qk_dn = (((2,), (2,)), ((0,), (0,)))
        av_dn = (((2,), (1,)), ((0,), (0,)))

        def partial(xb):
            qkv = lax.dot_general(xb, wqkv, proj_dn,
                                  preferred_element_type=jnp.float32)
            q = rope(qkv[:, :, :HD])
            k = rope(qkv[:, :, HD:2 * HD])
            v = qkv[:, :, 2 * HD:]
            acc = jnp.zeros((B, S, D), jnp.float32)
            for h in range(H):
                sl = slice(h * DH, (h + 1) * DH)
                s_ = lax.dot_general(q[:, :, sl], k[:, :, sl], qk_dn,
                                     preferred_element_type=jnp.float32)
                s_ = s_ * 0.125
                m = jnp.max(s_, axis=-1, keepdims=True)
                w = jnp.exp(s_ - m)
                w = w / jnp.sum(w, axis=-1, keepdims=True)
                ctx = lax.dot_general(w, v[:, :, sl], av_dn,
                                      preferred_element_type=jnp.float32)
                acc = acc + lax.dot_general(ctx, wo[sl, :], proj_dn,
                                            preferred_element_type=jnp.float32)
            return acc

        ag0 = ag_copy(0)
        ag0.start()
        pown_ref[...] = partial(x_ref[...])
        ag0.wait_recv()

        ag1 = ag_copy(1)
        ag1.start()
        sbuf_ref[0] = partial(xg_ref[0])
        rs0 = rs_copy(0)
        rs0.start()
        ag1.wait_recv()

        ag2 = ag_copy(2)
        ag2.start()
        tmp = partial(xg_ref[1])
        rs0.wait_recv()
        sbuf_ref[1] = tmp + rbuf_ref[0]
        rs1 = rs_copy(1)
        rs1.start()
        ag2.wait_recv()

        tmp = partial(xg_ref[2])
        rs1.wait_recv()
        sbuf_ref[2] = tmp + rbuf_ref[1]
        rs2 = rs_copy(2)
        rs2.start()
        rs2.wait_recv()
        out_ref[...] = rbuf_ref[2] + pown_ref[...]

        for r in (ag0, ag1, ag2, rs0, rs1, rs2):
            r.wait_send()

    return pl.pallas_call(
        body,
        out_shape=jax.ShapeDtypeStruct((B, S, D), jnp.float32),
        in_specs=[pl.BlockSpec(memory_space=pltpu.VMEM)] * 5,
        out_specs=pl.BlockSpec(memory_space=pltpu.VMEM),
        scratch_shapes=[
            pltpu.VMEM((N_DEV - 1, B, S, D), jnp.float32),
            pltpu.VMEM((B, S, D), jnp.float32),
            pltpu.VMEM((N_DEV - 1, B, S, D), jnp.float32),
            pltpu.VMEM((N_DEV - 1, B, S, D), jnp.float32),
            pltpu.SemaphoreType.DMA((N_DEV - 1,)),
            pltpu.SemaphoreType.DMA((N_DEV - 1,)),
            pltpu.SemaphoreType.DMA((N_DEV - 1,)),
            pltpu.SemaphoreType.DMA((N_DEV - 1,)),
        ],
        compiler_params=pltpu.CompilerParams(collective_id=0),
    )(x, Wq, Wk, Wv, Wo)


# device time: 47375 ns/iter; 1.3596x vs baseline; 1.0207x over previous
import jax
import jax.numpy as jnp
from jax import lax
from jax.experimental import pallas as pl
from jax.experimental.pallas import tpu as pltpu

N_DEV = 4
DH = 64


def kernel(x, Wq, Wk, Wv, Wo):
    B, S, D = x.shape
    H = Wq.shape[1] // DH
    HD = H * DH

    def body(x_ref, wq_ref, wk_ref, wv_ref, wo_ref, out_ref,
             xg_ref, pown_ref, sbuf_ref, rbuf_ref,
             ag_send, ag_recv, rs_send, rs_recv):
        my = lax.axis_index("i")
        left = lax.rem(my + N_DEV - 1, N_DEV)
        right = lax.rem(my + 1, N_DEV)

        barrier = pltpu.get_barrier_semaphore()
        for nbr in (left, right):
            pl.semaphore_signal(barrier, inc=1, device_id=(nbr,),
                                device_id_type=pl.DeviceIdType.MESH)
        pl.semaphore_wait(barrier, 2)

        def ag_copy(h):
            src = x_ref if h == 0 else xg_ref.at[h - 1]
            return pltpu.make_async_remote_copy(
                src_ref=src,
                dst_ref=xg_ref.at[h],
                send_sem=ag_send.at[h],
                recv_sem=ag_recv.at[h],
                device_id=(right,),
                device_id_type=pl.DeviceIdType.MESH,
            )

        def rs_copy(s):
            return pltpu.make_async_remote_copy(
                src_ref=sbuf_ref.at[s],
                dst_ref=rbuf_ref.at[s],
                send_sem=rs_send.at[s],
                recv_sem=rs_recv.at[s],
                device_id=(right,),
                device_id_type=pl.DeviceIdType.MESH,
            )

        posf = lax.broadcasted_iota(jnp.int32, (S, HD), 0).astype(jnp.float32)
        ji = lax.broadcasted_iota(jnp.int32, (S, HD), 1)
        jh = ji % DH
        k2f = ((jh // 2) * 2).astype(jnp.float32)
        inv = jnp.exp(-(k2f / DH) * jnp.log(10000.0))
        ang = posf * inv
        cos = jnp.cos(ang)
        sin = jnp.sin(ang)
        even = (ji % 2) == 0

        def rope(t):
            tl = jnp.concatenate([t[..., 1:], t[..., :1]], axis=-1)
            tr = jnp.concatenate([t[..., -1:], t[..., :-1]], axis=-1)
            t_r = jnp.where(even[None], -tl, tr)
            return t * cos[None] + t_r * sin[None]

        wqkv = jnp.concatenate([wq_ref[...], wk_ref[...], wv_ref[...]],
                               axis=1)
        wo = wo_ref[...]
        proj_dn = (((2,), (0,)), ((), ()))
        qk_dn = (((2,), (2,)), ((0,), (0,)))
        av_dn = (((2,), (1,)), ((0,), (0,)))

        bf = jnp.bfloat16
        wqkv_b = wqkv.astype(bf)
        wo_b = wo.astype(bf)

        def partial(xb):
            qkv = lax.dot_general(xb.astype(bf), wqkv_b, proj_dn,
                                  preferred_element_type=jnp.float32)
            q = rope(qkv[:, :, :HD]).astype(bf)
            k = rope(qkv[:, :, HD:2 * HD]).astype(bf)
            v = qkv[:, :, 2 * HD:].astype(bf)
            ctxs = []
            for h in range(H):
                sl = slice(h * DH, (h + 1) * DH)
                s_ = lax.dot_general(q[:, :, sl], k[:, :, sl], qk_dn,
                                     preferred_element_type=jnp.float32)
                s_ = s_ * 0.125
                m = jnp.max(s_, axis=-1, keepdims=True)
                w = jnp.exp(s_ - m)
                w = (w / jnp.sum(w, axis=-1, keepdims=True)).astype(bf)
                ctxs.append(lax.dot_general(w, v[:, :, sl], av_dn,
                                            preferred_element_type=jnp.float32))
            ctx = jnp.concatenate(ctxs, axis=-1).astype(bf)
            return lax.dot_general(ctx, wo_b, proj_dn,
                                   preferred_element_type=jnp.float32)

        ag0 = ag_copy(0)
        ag0.start()
        pown_ref[...] = partial(x_ref[...])
        ag0.wait_recv()

        ag1 = ag_copy(1)
        ag1.start()
        sbuf_ref[0] = partial(xg_ref[0])
        rs0 = rs_copy(0)
        rs0.start()
        ag1.wait_recv()

        ag2 = ag_copy(2)
        ag2.start()
        tmp = partial(xg_ref[1])
        rs0.wait_recv()
        sbuf_ref[1] = tmp + rbuf_ref[0]
        rs1 = rs_copy(1)
        rs1.start()
        ag2.wait_recv()

        tmp = partial(xg_ref[2])
        rs1.wait_recv()
        sbuf_ref[2] = tmp + rbuf_ref[1]
        rs2 = rs_copy(2)
        rs2.start()
        rs2.wait_recv()
        out_ref[...] = rbuf_ref[2] + pown_ref[...]

        for r in (ag0, ag1, ag2, rs0, rs1, rs2):
            r.wait_send()

    return pl.pallas_call(
        body,
        out_shape=jax.ShapeDtypeStruct((B, S, D), jnp.float32),
        in_specs=[pl.BlockSpec(memory_space=pltpu.VMEM)] * 5,
        out_specs=pl.BlockSpec(memory_space=pltpu.VMEM),
        scratch_shapes=[
            pltpu.VMEM((N_DEV - 1, B, S, D), jnp.float32),
            pltpu.VMEM((B, S, D), jnp.float32),
            pltpu.VMEM((N_DEV - 1, B, S, D), jnp.float32),
            pltpu.VMEM((N_DEV - 1, B, S, D), jnp.float32),
            pltpu.SemaphoreType.DMA((N_DEV - 1,)),
            pltpu.SemaphoreType.DMA((N_DEV - 1,)),
            pltpu.SemaphoreType.DMA((N_DEV - 1,)),
            pltpu.SemaphoreType.DMA((N_DEV - 1,)),
        ],
        compiler_params=pltpu.CompilerParams(collective_id=0),
    )(x, Wq, Wk, Wv, Wo)


# device time: 27026 ns/iter; 2.3833x vs baseline; 1.7529x over previous
import jax
import jax.numpy as jnp
from jax import lax
from jax.experimental import pallas as pl
from jax.experimental.pallas import tpu as pltpu

N_DEV = 4
DH = 64


def kernel(x, Wq, Wk, Wv, Wo):
    B, S, D = x.shape
    H = Wq.shape[1] // DH
    HD = H * DH
    bf = jnp.bfloat16

    def body(x_ref, wq_ref, wk_ref, wv_ref, wo_ref, out_ref,
             xsend_ref, xg_ref, pown_ref, psend_ref, prbuf_ref,
             ag_send, ag_recv, rs_send, rs_recv):
        my = lax.axis_index("i")

        barrier = pltpu.get_barrier_semaphore()
        for q in range(N_DEV - 1):
            peer = lax.rem(my + 1 + q, N_DEV)
            pl.semaphore_signal(barrier, inc=1, device_id=(peer,),
                                device_id_type=pl.DeviceIdType.MESH)
        pl.semaphore_wait(barrier, N_DEV - 1)

        def ag_copy(q):
            peer = lax.rem(my + 1 + q, N_DEV)
            return pltpu.make_async_remote_copy(
                src_ref=xsend_ref,
                dst_ref=xg_ref.at[2 - q],
                send_sem=ag_send.at[q],
                recv_sem=ag_recv.at[2 - q],
                device_id=(peer,),
                device_id_type=pl.DeviceIdType.MESH,
            )

        def ag_wait(q):
            return pltpu.make_async_remote_copy(
                src_ref=xsend_ref,
                dst_ref=xg_ref.at[q],
                send_sem=ag_send.at[q],
                recv_sem=ag_recv.at[q],
                device_id=(my,),
                device_id_type=pl.DeviceIdType.MESH,
            )

        def rs_copy(q):
            peer = lax.rem(my + 1 + q, N_DEV)
            return pltpu.make_async_remote_copy(
                src_ref=psend_ref.at[q],
                dst_ref=prbuf_ref.at[2 - q],
                send_sem=rs_send.at[q],
                recv_sem=rs_recv.at[2 - q],
                device_id=(peer,),
                device_id_type=pl.DeviceIdType.MESH,
            )

        def rs_wait(q):
            return pltpu.make_async_remote_copy(
                src_ref=psend_ref.at[q],
                dst_ref=prbuf_ref.at[q],
                send_sem=rs_send.at[q],
                recv_sem=rs_recv.at[q],
                device_id=(my,),
                device_id_type=pl.DeviceIdType.MESH,
            )

        posf = lax.broadcasted_iota(jnp.int32, (S, HD), 0).astype(jnp.float32)
        ji = lax.broadcasted_iota(jnp.int32, (S, HD), 1)
        jh = ji % DH
        k2f = ((jh // 2) * 2).astype(jnp.float32)
        inv = jnp.exp(-(k2f / DH) * jnp.log(10000.0))
        ang = posf * inv
        cos = jnp.cos(ang)
        sin = jnp.sin(ang)
        even = (ji % 2) == 0

        def rope(t):
            tl = jnp.concatenate([t[..., 1:], t[..., :1]], axis=-1)
            tr = jnp.concatenate([t[..., -1:], t[..., :-1]], axis=-1)
            t_r = jnp.where(even[None], -tl, tr)
            return t * cos[None] + t_r * sin[None]

        wqkv_b = jnp.concatenate(
            [wq_ref[...], wk_ref[...], wv_ref[...]], axis=1).astype(bf)
        wo_b = wo_ref[...].astype(bf)
        proj_dn = (((2,), (0,)), ((), ()))
        qk_dn = (((2,), (2,)), ((0,), (0,)))
        av_dn = (((2,), (1,)), ((0,), (0,)))

        def partial(xb_bf):
            qkv = lax.dot_general(xb_bf, wqkv_b, proj_dn,
                                  preferred_element_type=jnp.float32)
            q = rope(qkv[:, :, :HD]).astype(bf)
            k = rope(qkv[:, :, HD:2 * HD]).astype(bf)
            v = qkv[:, :, 2 * HD:].astype(bf)
            ctxs = []
            for h in range(H):
                sl = slice(h * DH, (h + 1) * DH)
                s_ = lax.dot_general(q[:, :, sl], k[:, :, sl], qk_dn,
                                     preferred_element_type=jnp.float32)
                s_ = s_ * 0.125
                m = jnp.max(s_, axis=-1, keepdims=True)
                w = jnp.exp(s_ - m)
                w = (w / jnp.sum(w, axis=-1, keepdims=True)).astype(bf)
                ctxs.append(lax.dot_general(w, v[:, :, sl], av_dn,
                                            preferred_element_type=jnp.float32))
            ctx = jnp.concatenate(ctxs, axis=-1).astype(bf)
            return lax.dot_general(ctx, wo_b, proj_dn,
                                   preferred_element_type=jnp.float32)

        xsend_ref[...] = x_ref[...].astype(bf)
        ags = {}
        for q in (0, 2, 1):
            ags[q] = ag_copy(q)
            ags[q].start()

        pown_ref[...] = partial(x_ref[...].astype(bf))

        rss = {}
        for q in (0, 2, 1):
            ag_wait(q).wait_recv()
            psend_ref[q] = partial(xg_ref[q]).astype(bf)
            rss[q] = rs_copy(q)
            rss[q].start()

        acc = pown_ref[...]
        for q in (0, 2, 1):
            rs_wait(q).wait_recv()
            acc = acc + prbuf_ref[q].astype(jnp.float32)
        out_ref[...] = acc

        for q in range(N_DEV - 1):
            ags[q].wait_send()
            rss[q].wait_send()

    return pl.pallas_call(
        body,
        out_shape=jax.ShapeDtypeStruct((B, S, D), jnp.float32),
        in_specs=[pl.BlockSpec(memory_space=pltpu.VMEM)] * 5,
        out_specs=pl.BlockSpec(memory_space=pltpu.VMEM),
        scratch_shapes=[
            pltpu.VMEM((B, S, D), bf),
            pltpu.VMEM((N_DEV - 1, B, S, D), bf),
            pltpu.VMEM((B, S, D), jnp.float32),
            pltpu.VMEM((N_DEV - 1, B, S, D), bf),
            pltpu.VMEM((N_DEV - 1, B, S, D), bf),
            pltpu.SemaphoreType.DMA((N_DEV - 1,)),
            pltpu.SemaphoreType.DMA((N_DEV - 1,)),
            pltpu.SemaphoreType.DMA((N_DEV - 1,)),
            pltpu.SemaphoreType.DMA((N_DEV - 1,)),
        ],
        compiler_params=pltpu.CompilerParams(collective_id=0),
    )(x, Wq, Wk, Wv, Wo)
